# baseline (speedup 1.0000x reference)
import functools

import jax
import jax.numpy as jnp
from jax import lax
from jax.experimental import pallas as pl
from jax.experimental.pallas import tpu as pltpu

try:
    from jax.experimental.pallas import tpu_sc as plsc
    _HAS_SC = True
except ImportError:
    _HAS_SC = False

B = 4096
D = 16
N_USERS = 100000
N_COURSES = 1000
TOPK = 6

ROW_BLOCK = 1024
N_BLOCKS = B // ROW_BLOCK


def _topk_body(emb_blk_ref, emb_full_ref, vals_ref, idx_ref):
    emb_full = emb_full_ref[...]
    norms = jnp.sqrt(jnp.sum(emb_full * emb_full, axis=1, keepdims=True))
    normed_full = emb_full / norms

    emb_blk = emb_blk_ref[...]
    nb = jnp.sqrt(jnp.sum(emb_blk * emb_blk, axis=1, keepdims=True))
    normed_blk = emb_blk / nb

    c = lax.dot_general(
        normed_blk, normed_full,
        dimension_numbers=(((1,), (1,)), ((), ())),
        preferred_element_type=jnp.float32,
    )

    col = lax.broadcasted_iota(jnp.int32, (ROW_BLOCK, B), 1)
    neg = jnp.float32(-jnp.inf)
    vals = []
    idxs = []
    for _ in range(TOPK):
        m = jnp.max(c, axis=1)
        i = jnp.argmax(c, axis=1).astype(jnp.int32)
        vals.append(m)
        idxs.append(i)
        c = jnp.where(col == i[:, None], neg, c)

    zf = jnp.zeros((ROW_BLOCK,), jnp.float32)
    zi = jnp.zeros((ROW_BLOCK,), jnp.int32)
    vals_ref[...] = jnp.stack(vals + [zf, zf])
    idx_ref[...] = jnp.stack(idxs + [zi, zi])


def _topk_stage(users_embeddings):
    grid = (N_BLOCKS,)
    vals8, idx8 = pl.pallas_call(
        _topk_body,
        grid=grid,
        in_specs=[
            pl.BlockSpec((ROW_BLOCK, D), lambda i: (i, 0)),
            pl.BlockSpec((B, D), lambda i: (0, 0)),
        ],
        out_specs=[
            pl.BlockSpec((8, ROW_BLOCK), lambda i: (0, i)),
            pl.BlockSpec((8, ROW_BLOCK), lambda i: (0, i)),
        ],
        out_shape=[
            jax.ShapeDtypeStruct((8, B), jnp.float32),
            jax.ShapeDtypeStruct((8, B), jnp.int32),
        ],
    )(users_embeddings, users_embeddings)
    return vals8, idx8


_NC = 2
_NS = 16
_NW = _NC * _NS
_RPT = B // _NW
_L = 16


_WPR = 256


def _sc_neighbors_stage(idx8, n_users):
    mesh = plsc.VectorSubcoreMesh(core_axis_name="c", subcore_axis_name="s")

    @functools.partial(
        pl.kernel,
        mesh=mesh,
        out_type=jax.ShapeDtypeStruct((TOPK, B), jnp.int32),
        compiler_params=pltpu.CompilerParams(needs_layout_passes=False),
        scratch_types=[
            pltpu.VMEM((B,), jnp.int32),
            pltpu.VMEM((TOPK, _RPT), jnp.int32),
            pltpu.VMEM((TOPK, _RPT), jnp.int32),
        ],
    )
    def nbr_kernel(idx_hbm, nu_hbm, out_hbm, nu_v, idx_v, nbr_v):
        wid = lax.axis_index("s") * _NC + lax.axis_index("c")
        base = wid * _RPT

        pltpu.sync_copy(nu_hbm, nu_v)
        for j in range(TOPK):
            pltpu.sync_copy(idx_hbm.at[j, pl.ds(base, _RPT)], idx_v.at[j])
        for j in range(TOPK):
            for g in range(_RPT // _L):
                sl = pl.ds(g * _L, _L)
                nbr_v[j, sl] = plsc.load_gather(nu_v, [idx_v[j, sl]])
        for j in range(TOPK):
            pltpu.sync_copy(nbr_v.at[j], out_hbm.at[j, pl.ds(base, _RPT)])

    return nbr_kernel(idx8, n_users)


def _sc_weighted_sum_stage(vals8, chosen_f):
    mesh = plsc.VectorSubcoreMesh(core_axis_name="c", subcore_axis_name="s")

    @functools.partial(
        pl.kernel,
        mesh=mesh,
        out_type=jax.ShapeDtypeStruct((B,), jnp.float32),
        compiler_params=pltpu.CompilerParams(needs_layout_passes=False),
        scratch_types=[
            pltpu.VMEM((TOPK, _RPT), jnp.float32),
            pltpu.VMEM((TOPK, _RPT), jnp.float32),
            pltpu.VMEM((_RPT,), jnp.float32),
        ],
    )
    def ws_kernel(vals_hbm, ch_hbm, out_hbm, vals_v, ch_v, acc_v):
        wid = lax.axis_index("s") * _NC + lax.axis_index("c")
        base = wid * _RPT

        for j in range(TOPK):
            pltpu.sync_copy(vals_hbm.at[j, pl.ds(base, _RPT)], vals_v.at[j])
            pltpu.sync_copy(ch_hbm.at[j, pl.ds(base, _RPT)], ch_v.at[j])
        for g in range(_RPT // _L):
            sl = pl.ds(g * _L, _L)
            acc = vals_v[0, sl] * ch_v[0, sl]
            for j in range(1, TOPK):
                acc = acc + vals_v[j, sl] * ch_v[j, sl]
            acc_v[sl] = acc / jnp.float32(6.0)
        pltpu.sync_copy(acc_v, out_hbm.at[pl.ds(base, _RPT)])

    return ws_kernel(vals8, chosen_f)


def kernel(users_embeddings, interactions, n_users, n_entitys, course):
    vals8, idx8 = _topk_stage(users_embeddings)
    nbr = _sc_neighbors_stage(idx8, n_users)
    chosen_f = interactions[nbr, n_entitys[None, :]].astype(jnp.float32)
    return _sc_weighted_sum_stage(vals8, chosen_f)


# speedup vs baseline: 1.0163x; 1.0163x over previous
import functools

import jax
import jax.numpy as jnp
from jax import lax
from jax.experimental import pallas as pl
from jax.experimental.pallas import tpu as pltpu

try:
    from jax.experimental.pallas import tpu_sc as plsc
    _HAS_SC = True
except ImportError:
    _HAS_SC = False

B = 4096
D = 16
N_USERS = 100000
N_COURSES = 1000
TOPK = 6

ROW_BLOCK = 1024
N_BLOCKS = B // ROW_BLOCK


def _topk_body(emb_blk_ref, emb_full_ref, vals_ref, idx_ref):
    emb_full = emb_full_ref[...]
    norms = jnp.sqrt(jnp.sum(emb_full * emb_full, axis=1, keepdims=True))
    normed_full = emb_full / norms

    emb_blk = emb_blk_ref[...]
    nb = jnp.sqrt(jnp.sum(emb_blk * emb_blk, axis=1, keepdims=True))
    normed_blk = emb_blk / nb

    c = lax.dot_general(
        normed_blk, normed_full,
        dimension_numbers=(((1,), (1,)), ((), ())),
        preferred_element_type=jnp.float32,
    )

    col = lax.broadcasted_iota(jnp.int32, (ROW_BLOCK, B), 1)
    neg = jnp.float32(-jnp.inf)
    vals = []
    idxs = []
    for _ in range(TOPK):
        m = jnp.max(c, axis=1)
        eq = c == m[:, None]
        i = jnp.min(jnp.where(eq, col, B), axis=1)
        vals.append(m)
        idxs.append(i)
        c = jnp.where(col == i[:, None], neg, c)

    zf = jnp.zeros((ROW_BLOCK,), jnp.float32)
    zi = jnp.zeros((ROW_BLOCK,), jnp.int32)
    vals_ref[...] = jnp.stack(vals + [zf, zf])
    idx_ref[...] = jnp.stack(idxs + [zi, zi])


def _topk_stage(users_embeddings):
    grid = (N_BLOCKS,)
    vals8, idx8 = pl.pallas_call(
        _topk_body,
        grid=grid,
        in_specs=[
            pl.BlockSpec((ROW_BLOCK, D), lambda i: (i, 0)),
            pl.BlockSpec((B, D), lambda i: (0, 0)),
        ],
        out_specs=[
            pl.BlockSpec((8, ROW_BLOCK), lambda i: (0, i)),
            pl.BlockSpec((8, ROW_BLOCK), lambda i: (0, i)),
        ],
        out_shape=[
            jax.ShapeDtypeStruct((8, B), jnp.float32),
            jax.ShapeDtypeStruct((8, B), jnp.int32),
        ],
    )(users_embeddings, users_embeddings)
    return vals8, idx8


_NC = 2
_NS = 16
_NW = _NC * _NS
_RPT = B // _NW
_L = 16


def _sc_neighbors_stage(idx8, n_users):
    mesh = plsc.VectorSubcoreMesh(core_axis_name="c", subcore_axis_name="s")

    @functools.partial(
        pl.kernel,
        mesh=mesh,
        out_type=jax.ShapeDtypeStruct((TOPK, B), jnp.int32),
        compiler_params=pltpu.CompilerParams(needs_layout_passes=False),
        scratch_types=[
            pltpu.VMEM((B,), jnp.int32),
            pltpu.VMEM((TOPK, _RPT), jnp.int32),
            pltpu.VMEM((TOPK, _RPT), jnp.int32),
        ],
    )
    def nbr_kernel(idx_hbm, nu_hbm, out_hbm, nu_v, idx_v, nbr_v):
        wid = lax.axis_index("s") * _NC + lax.axis_index("c")
        base = wid * _RPT

        pltpu.sync_copy(nu_hbm, nu_v)
        for j in range(TOPK):
            pltpu.sync_copy(idx_hbm.at[j, pl.ds(base, _RPT)], idx_v.at[j])
        for j in range(TOPK):
            for g in range(_RPT // _L):
                sl = pl.ds(g * _L, _L)
                nbr_v[j, sl] = plsc.load_gather(nu_v, [idx_v[j, sl]])
        for j in range(TOPK):
            pltpu.sync_copy(nbr_v.at[j], out_hbm.at[j, pl.ds(base, _RPT)])

    return nbr_kernel(idx8, n_users)


def _sc_weighted_sum_stage(vals8, chosen_f):
    mesh = plsc.VectorSubcoreMesh(core_axis_name="c", subcore_axis_name="s")

    @functools.partial(
        pl.kernel,
        mesh=mesh,
        out_type=jax.ShapeDtypeStruct((B,), jnp.float32),
        compiler_params=pltpu.CompilerParams(needs_layout_passes=False),
        scratch_types=[
            pltpu.VMEM((TOPK, _RPT), jnp.float32),
            pltpu.VMEM((TOPK, _RPT), jnp.float32),
            pltpu.VMEM((_RPT,), jnp.float32),
        ],
    )
    def ws_kernel(vals_hbm, ch_hbm, out_hbm, vals_v, ch_v, acc_v):
        wid = lax.axis_index("s") * _NC + lax.axis_index("c")
        base = wid * _RPT

        for j in range(TOPK):
            pltpu.sync_copy(vals_hbm.at[j, pl.ds(base, _RPT)], vals_v.at[j])
            pltpu.sync_copy(ch_hbm.at[j, pl.ds(base, _RPT)], ch_v.at[j])
        for g in range(_RPT // _L):
            sl = pl.ds(g * _L, _L)
            acc = vals_v[0, sl] * ch_v[0, sl]
            for j in range(1, TOPK):
                acc = acc + vals_v[j, sl] * ch_v[j, sl]
            acc_v[sl] = acc / jnp.float32(6.0)
        pltpu.sync_copy(acc_v, out_hbm.at[pl.ds(base, _RPT)])

    return ws_kernel(vals8, chosen_f)


def kernel(users_embeddings, interactions, n_users, n_entitys, course):
    vals8, idx8 = _topk_stage(users_embeddings)
    nbr = _sc_neighbors_stage(idx8, n_users)
    chosen_f = interactions[nbr, n_entitys[None, :]].astype(jnp.float32)
    return _sc_weighted_sum_stage(vals8, chosen_f)
